# Initial kernel scaffold; baseline (speedup 1.0000x reference)
import jax
import jax.numpy as jnp
from jax.experimental import pallas as pl


def kernel(nodes, edges, senders, receivers, params):
    raise NotImplementedError("write your pallas kernel here")


# speedup vs baseline: 2.8447x; 2.8447x over previous
import functools

import jax
import jax.numpy as jnp
from jax import lax
from jax.experimental import pallas as pl
from jax.experimental.pallas import tpu as pltpu
from jax.experimental.pallas import tpu_sc as plsc

_N = 10000
_E = 320000
_D = 128
_NC = 2
_NS = 16
_NW = _NC * _NS
_EPW = _E // _NW
_C = 80
_IPW = _EPW // _C
_NPAD = 10240
_RPT = _NPAD // _NS

_NBLK = 1000
_EBLK = 1600


def _dot(x, w):
    return lax.dot_general(x, w, (((1,), (0,)), ((), ())),
                           preferred_element_type=jnp.float32)


def _ln(x, g, beta):
    mu = jnp.mean(x, axis=-1, keepdims=True)
    var = jnp.mean((x - mu) ** 2, axis=-1, keepdims=True)
    return (x - mu) * lax.rsqrt(var + 1e-5) * g + beta


def _hidden(h, w, b, g, beta):
    return jax.nn.relu(_ln(_dot(h, w) + b, g, beta))


def _full(shape):
    return pl.BlockSpec(shape, lambda i: tuple(0 for _ in shape))


def _rows(blk, d):
    return pl.BlockSpec((blk, d), lambda i: (i, 0))


def _prep_mlp(p):
    out = []
    for lyr in p:
        out.append(lyr['W'])
        out.append(lyr['b'][None, :])
        if 'g' in lyr:
            out.append(lyr['g'][None, :])
            out.append(lyr['beta'][None, :])
    return out


def _mlp_ln_body(x, w1, b1, g1, be1, w2, b2, g2, be2, w3, b3, o):
    h = _hidden(x[...], w1[...], b1[...], g1[...], be1[...])
    h = _hidden(h, w2[...], b2[...], g2[...], be2[...])
    o[...] = _dot(h, w3[...]) + b3[...]


def _enc_mlp(x, p, blk):
    n, din = x.shape
    ws = _prep_mlp(p)
    return pl.pallas_call(
        _mlp_ln_body,
        grid=(n // blk,),
        in_specs=[_rows(blk, din)] + [_full(w.shape) for w in ws],
        out_specs=_rows(blk, _D),
        out_shape=jax.ShapeDtypeStruct((n, _D), jnp.float32),
    )(x, *ws)


def _edge_body(want_e, sg, rg, e, ws, wr, we, b1, g1, be1,
               w2, b2, g2, be2, w3, b3, ue_ref, *maybe_en):
    h = _dot(sg[...], ws[...]) + _dot(rg[...], wr[...]) + _dot(e[...], we[...]) + b1[...]
    h = jax.nn.relu(_ln(h, g1[...], be1[...]))
    h = _hidden(h, w2[...], b2[...], g2[...], be2[...])
    ue = _dot(h, w3[...]) + b3[...]
    ue_ref[...] = ue
    if want_e:
        maybe_en[0][...] = e[...] + ue


def _edge_mlp(sg, rg, e, p, want_e):
    w1 = p[0]['W']
    ws_ = [w1[:_D], w1[_D:2 * _D], w1[2 * _D:],
           p[0]['b'][None, :], p[0]['g'][None, :], p[0]['beta'][None, :],
           p[1]['W'], p[1]['b'][None, :], p[1]['g'][None, :], p[1]['beta'][None, :],
           p[2]['W'], p[2]['b'][None, :]]
    out_shape = [jax.ShapeDtypeStruct((_E, _D), jnp.float32)]
    out_specs = [_rows(_EBLK, _D)]
    if want_e:
        out_shape.append(jax.ShapeDtypeStruct((_E, _D), jnp.float32))
        out_specs.append(_rows(_EBLK, _D))
    res = pl.pallas_call(
        functools.partial(_edge_body, want_e),
        grid=(_E // _EBLK,),
        in_specs=[_rows(_EBLK, _D)] * 3 + [_full(w.shape) for w in ws_],
        out_specs=out_specs,
        out_shape=out_shape,
    )(sg, rg, e, *ws_)
    return res if want_e else res[0]


def _node_body(n, a0, a1, wn, wa, b1, g1, be1, w2, b2, g2, be2, w3, b3, o):
    h = _dot(n[...], wn[...]) + _dot(a0[...] + a1[...], wa[...]) + b1[...]
    h = jax.nn.relu(_ln(h, g1[...], be1[...]))
    h = _hidden(h, w2[...], b2[...], g2[...], be2[...])
    o[...] = n[...] + _dot(h, w3[...]) + b3[...]


def _node_mlp(n, a0, a1, p):
    w1 = p[0]['W']
    ws_ = [w1[:_D], w1[_D:],
           p[0]['b'][None, :], p[0]['g'][None, :], p[0]['beta'][None, :],
           p[1]['W'], p[1]['b'][None, :], p[1]['g'][None, :], p[1]['beta'][None, :],
           p[2]['W'], p[2]['b'][None, :]]
    return pl.pallas_call(
        _node_body,
        grid=(_N // _NBLK,),
        in_specs=[_rows(_NBLK, _D)] * 3 + [_full(w.shape) for w in ws_],
        out_specs=_rows(_NBLK, _D),
        out_shape=jax.ShapeDtypeStruct((_N, _D), jnp.float32),
    )(n, a0, a1, *ws_)


def _dec_body(x, w1, b1, w2, b2, w3, b3, o):
    h = jax.nn.relu(_dot(x[...], w1[...]) + b1[...])
    h = jax.nn.relu(_dot(h, w2[...]) + b2[...])
    o[...] = _dot(h, w3[...]) + b3[...]


def _dec_mlp(x, p):
    w3 = jnp.pad(p[2]['W'], ((0, 0), (0, _D - p[2]['W'].shape[1])))
    b3 = jnp.pad(p[2]['b'], (0, _D - p[2]['b'].shape[0]))[None, :]
    ws_ = [p[0]['W'], p[0]['b'][None, :], p[1]['W'], p[1]['b'][None, :], w3, b3]
    out = pl.pallas_call(
        _dec_body,
        grid=(_N // _NBLK,),
        in_specs=[_rows(_NBLK, _D)] + [_full(w.shape) for w in ws_],
        out_specs=_rows(_NBLK, _D),
        out_shape=jax.ShapeDtypeStruct((_N, _D), jnp.float32),
    )(x, *ws_)
    return out[:, :p[2]['W'].shape[1]]


@functools.cache
def _gather_fn():
    mesh = plsc.VectorSubcoreMesh(core_axis_name="c", subcore_axis_name="s",
                                  num_cores=_NC, num_subcores=_NS)

    @functools.partial(
        pl.kernel,
        out_type=(jax.ShapeDtypeStruct((_E, _D), jnp.float32),
                  jax.ShapeDtypeStruct((_E, _D), jnp.float32)),
        mesh=mesh,
        scratch_types=[
            pltpu.VMEM((_IPW, _C), jnp.int32),
            pltpu.VMEM((_IPW, _C), jnp.int32),
            pltpu.VMEM((_C, _D), jnp.float32),
            pltpu.VMEM((_C, _D), jnp.float32),
            pltpu.SemaphoreType.DMA,
            pltpu.SemaphoreType.DMA,
        ],
    )
    def _sc_gather(n_hbm, s_hbm, r_hbm, sg_hbm, rg_hbm,
                   sidx, ridx, sbuf, rbuf, sem1, sem2):
        wid = lax.axis_index("s") * _NC + lax.axis_index("c")
        ebase = wid * _EPW
        pltpu.sync_copy(s_hbm.at[wid], sidx)
        pltpu.sync_copy(r_hbm.at[wid], ridx)

        def body(j, carry):
            off = ebase + j * _C
            cp1 = pltpu.async_copy(n_hbm.at[sidx.at[j]], sbuf, sem1)
            cp2 = pltpu.async_copy(n_hbm.at[ridx.at[j]], rbuf, sem2)
            cp1.wait()
            cp2.wait()
            pltpu.sync_copy(sbuf, sg_hbm.at[pl.ds(off, _C)])
            pltpu.sync_copy(rbuf, rg_hbm.at[pl.ds(off, _C)])
            return carry

        lax.fori_loop(0, _IPW, body, 0)

    return _sc_gather


@functools.cache
def _scatter_fn():
    mesh = plsc.VectorSubcoreMesh(core_axis_name="c", subcore_axis_name="s",
                                  num_cores=_NC, num_subcores=_NS)

    @functools.partial(
        pl.kernel,
        out_type=jax.ShapeDtypeStruct((_NC, _NPAD, _D), jnp.float32),
        mesh=mesh,
        scratch_types=[
            pltpu.VMEM((_IPW, _C), jnp.int32),
            pltpu.VMEM((_C, _D), jnp.float32),
            pltpu.VMEM_SHARED((_NPAD, _D), jnp.float32),
            pltpu.SemaphoreType.DMA,
        ],
    )
    def _sc_scatter(ue_hbm, r_hbm, out_hbm, ridx, ebuf, acc, sem):
        cid = lax.axis_index("c")
        sid = lax.axis_index("s")
        wid = sid * _NC + cid

        def zrow(i, carry):
            for kk in range(_D // 16):
                ebuf[i, pl.ds(kk * 16, 16)] = jnp.zeros((16,), jnp.float32)
            return carry

        lax.fori_loop(0, _C, zrow, 0)
        for z in range(_RPT // _C):
            pltpu.sync_copy(ebuf, acc.at[pl.ds(sid * _RPT + z * _C, _C)])
        plsc.subcore_barrier()

        ebase = wid * _EPW
        pltpu.sync_copy(r_hbm.at[wid], ridx)

        def body(j, carry):
            pltpu.sync_copy(ue_hbm.at[pl.ds(ebase + j * _C, _C)], ebuf)
            pltpu.sync_copy(ebuf, acc.at[ridx.at[j]], add=True)
            return carry

        lax.fori_loop(0, _IPW, body, 0)
        plsc.subcore_barrier()
        pltpu.sync_copy(acc.at[pl.ds(sid * _RPT, _RPT)],
                        out_hbm.at[cid, pl.ds(sid * _RPT, _RPT)])

    return _sc_scatter


def kernel(nodes, edges, senders, receivers, params):
    send2d = senders.astype(jnp.int32).reshape(_NW, _IPW, _C)
    recv2d = receivers.astype(jnp.int32).reshape(_NW, _IPW, _C)

    n = _enc_mlp(nodes, params['enc_node'], _NBLK)
    e = _enc_mlp(edges, params['enc_edge'], _EBLK)

    for si, step in enumerate(params['proc']):
        sg, rg = _gather_fn()(n, send2d, recv2d)
        if si == 0:
            ue, e = _edge_mlp(sg, rg, e, step['edge'], want_e=True)
        else:
            ue = _edge_mlp(sg, rg, e, step['edge'], want_e=False)
        agg = _scatter_fn()(ue, recv2d)
        n = _node_mlp(n, agg[0, :_N], agg[1, :_N], step['node'])

    return _dec_mlp(n, params['dec'])
